# Initial kernel scaffold; baseline (speedup 1.0000x reference)
import jax
import jax.numpy as jnp
from jax.experimental import pallas as pl


def kernel(X, cof0, cof1):
    raise NotImplementedError("write your pallas kernel here")


# speedup vs baseline: 1.1569x; 1.1569x over previous
import functools

import jax
import jax.numpy as jnp
from jax import lax
from jax.experimental import pallas as pl
from jax.experimental.pallas import tpu as pltpu
from jax.experimental.pallas import tpu_sc as plsc

B, H, W = 4, 512, 512
HW = H * W
NC, NS, L = 2, 16, 16
NW = NC * NS
TOTAL = 65536
PER_W = TOTAL // NW
CHUNK = 128
NCHUNK = PER_W // CHUNK
PER_BATCH = TOTAL // B
W_PER_B = NW // B


def _sc_gather(cof_hbm, x_hbm, out_hbm, idx_v, vals_v, sem):
    wid = lax.axis_index("s") * NC + lax.axis_index("c")
    base = (wid // W_PER_B) * HW

    pltpu.sync_copy(cof_hbm.at[wid], idx_v)

    for j in range(NCHUNK):
        for i in range(CHUNK // L):
            idx_v[j, pl.ds(i * L, L)] = idx_v[j, pl.ds(i * L, L)] + base

    copies = [
        pltpu.async_copy(x_hbm.at[idx_v.at[j]], vals_v.at[j], sem)
        for j in range(NCHUNK)
    ]
    for c in copies:
        c.wait()

    pltpu.sync_copy(vals_v, out_hbm.at[wid])


@jax.jit
def kernel(X, cof0, cof1):
    b = X.shape[0]
    xflat = X.reshape(-1)
    cof = jnp.concatenate(
        [cof0.reshape(b, -1), cof1.reshape(b, -1)], axis=1
    ).astype(jnp.int32).reshape(NW, NCHUNK, CHUNK)

    mesh = plsc.VectorSubcoreMesh(core_axis_name="c", subcore_axis_name="s")
    run = functools.partial(
        pl.kernel,
        mesh=mesh,
        out_type=jax.ShapeDtypeStruct((NW, NCHUNK, CHUNK), jnp.float32),
        scratch_types=[
            pltpu.VMEM((NCHUNK, CHUNK), jnp.int32),
            pltpu.VMEM((NCHUNK, CHUNK), jnp.float32),
            pltpu.SemaphoreType.DMA,
        ],
    )(_sc_gather)
    out = run(cof, xflat)
    return out.reshape(b, -1, 2)


# speedup vs baseline: 1.4056x; 1.2150x over previous
import functools

import jax
import jax.numpy as jnp
from jax import lax
from jax.experimental import pallas as pl
from jax.experimental.pallas import tpu as pltpu
from jax.experimental.pallas import tpu_sc as plsc

B, H, W = 4, 512, 512
HW = H * W
NC, NS, L = 2, 16, 16
NW = NC * NS
NPAIR = 8192
PAIRS_W = 1024
NVAL = 2 * PAIRS_W
CHUNK = 128
NCHUNK = NVAL // CHUNK
BLK = 128
NBLK = PAIRS_W // BLK


def _sc_gather(g0_hbm, g1_hbm, x_hbm, out_hbm, idx_v, vals_v, vi_v, sem):
    wid = lax.axis_index("s") * NC + lax.axis_index("c")
    half = wid // 16
    w2 = wid % 16
    b = w2 // 4
    q = w2 % 4

    @pl.when(half == 0)
    def _():
        pltpu.sync_copy(g0_hbm.at[b, q], idx_v)

    @pl.when(half == 1)
    def _():
        pltpu.sync_copy(g1_hbm.at[b, q], idx_v)

    copies = [
        pltpu.async_copy(
            x_hbm.at[idx_v.at[pl.ds(j * CHUNK, CHUNK)]],
            vals_v.at[pl.ds(j * CHUNK, CHUNK)],
            sem,
        )
        for j in range(NCHUNK)
    ]
    for c in copies:
        c.wait()

    lane = lax.iota(jnp.int32, L)
    rows0 = lax.shift_right_logical(lane, 1)
    cols = lax.bitwise_and(lane, 1)
    pair0 = half * (NPAIR // 2) + q * PAIRS_W

    def _block(blk, carry):
        for i in range(2 * BLK // L):
            v = vals_v[pl.ds(blk * 2 * BLK + i * L, L)]
            plsc.store_scatter(vi_v, [rows0 + i * (L // 2), cols], v)
        pltpu.sync_copy(vi_v, out_hbm.at[b, pl.ds(pair0 + blk * BLK, BLK), :])
        return carry

    lax.fori_loop(0, NBLK, _block, 0)


@jax.jit
def kernel(X, cof0, cof1):
    b = X.shape[0]
    xflat = X.reshape(-1)
    base = (jnp.arange(b, dtype=jnp.int32) * HW)[:, None, None]
    g0 = (cof0.astype(jnp.int32) + base).reshape(b, 4, NVAL)
    g1 = (cof1.astype(jnp.int32) + base).reshape(b, 4, NVAL)

    mesh = plsc.VectorSubcoreMesh(core_axis_name="c", subcore_axis_name="s")
    run = functools.partial(
        pl.kernel,
        mesh=mesh,
        compiler_params=pltpu.CompilerParams(needs_layout_passes=False),
        out_type=jax.ShapeDtypeStruct((b, NPAIR, 2), jnp.float32),
        scratch_types=[
            pltpu.VMEM((NVAL,), jnp.int32),
            pltpu.VMEM((NVAL,), jnp.float32),
            pltpu.VMEM((BLK, 2), jnp.float32),
            pltpu.SemaphoreType.DMA,
        ],
    )(_sc_gather)
    return run(g0, g1, xflat)
